# baseline (speedup 1.0000x reference)
import functools
import math

import jax
import jax.numpy as jnp
import numpy as np
from jax import lax
from jax.experimental import pallas as pl
from jax.experimental.pallas import tpu as pltpu
from jax.experimental.pallas import tpu_sc as plsc

D = 128
SEQ = 200
NBUF = 2
SCALE = math.sqrt(float(D))


def _positional_encoding(length, depth):
    half = depth // 2
    positions = np.arange(length)[:, None].astype(np.float32)
    depths = np.arange(half)[None, :].astype(np.float32) / float(half)
    angle_rates = 1.0 / (10000.0 ** depths)
    angle_rads = positions * angle_rates
    return np.concatenate([np.sin(angle_rads), np.cos(angle_rads)], axis=-1)


_PE = jnp.asarray(_positional_encoding(2048, D)[:SEQ], dtype=jnp.float32)


@functools.cache
def _make_kernel(n_batch):
    info = plsc.get_sparse_core_info()
    nc, ns = info.num_cores, info.num_subcores
    nw = nc * ns
    spw = n_batch // nw
    mesh = plsc.VectorSubcoreMesh(core_axis_name="c", subcore_axis_name="s")
    n_groups = (spw - 2) // NBUF

    @functools.partial(
        pl.kernel,
        out_type=jax.ShapeDtypeStruct((n_batch, SEQ, D), jnp.float32),
        mesh=mesh,
        scratch_types=[
            pltpu.VMEM((spw * SEQ,), jnp.int32),
            pltpu.VMEM((SEQ, D), jnp.float32),
        ] + [pltpu.VMEM((SEQ, D), jnp.float32)] * NBUF
          + [pltpu.SemaphoreType.DMA] * (2 * NBUF),
    )
    def k(x_hbm, table_hbm, pe_hbm, out_hbm, idx_v, pe_v,
          r0, r1, g0, g1, o0, o1):
        rows = (r0, r1)
        gsem = (g0, g1)
        osem = (o0, o1)
        wid = lax.axis_index("s") * nc + lax.axis_index("c")
        pltpu.sync_copy(x_hbm.at[pl.ds(wid * spw * SEQ, spw * SEQ)], idx_v)
        pltpu.sync_copy(pe_hbm, pe_v)
        row_base = wid * spw * SEQ

        def gather(u, b):
            return pltpu.make_async_copy(
                table_hbm.at[idx_v.at[pl.ds(u * SEQ, SEQ)]], rows[b],
                gsem[b])

        def out_cp(u, b):
            return pltpu.make_async_copy(
                rows[b], out_hbm.at[pl.ds(row_base + u * SEQ, SEQ)],
                osem[b])

        def compute(b):
            buf = rows[b]

            def row_body(t, c):
                for g in range(D // 16):
                    sl = pl.ds(g * 16, 16)
                    buf[t, sl] = buf[t, sl] * SCALE + pe_v[t, sl]
                return c

            lax.fori_loop(0, SEQ, row_body, 0, unroll=2)

        gather(0, 0).start()
        gather(1, 1).start()

        def group(p, carry):
            for j in range(2):
                u = 2 * p + j
                gather(u, j).wait()
                compute(j)
                pltpu.sync_copy(rows[j], out_hbm.at[wid * spw + u])

                @pl.when(u + 2 < spw)
                def _():
                    gather(u + 2, j).start()
            return carry

        lax.fori_loop(0, spw // 2, group, 0)

    return k


def kernel(x, table):
    n_batch, seq = x.shape
    out = _make_kernel(n_batch)(x.reshape(-1), table, _PE)
    return out.reshape(n_batch, seq, D)


# speedup vs baseline: 1.6633x; 1.6633x over previous
import functools
import math

import jax
import jax.numpy as jnp
import numpy as np
from jax import lax
from jax.experimental import pallas as pl
from jax.experimental.pallas import tpu as pltpu
from jax.experimental.pallas import tpu_sc as plsc

D = 128
SEQ = 200
NBUF = 2
SCALE = math.sqrt(float(D))


def _positional_encoding(length, depth):
    half = depth // 2
    positions = np.arange(length)[:, None].astype(np.float32)
    depths = np.arange(half)[None, :].astype(np.float32) / float(half)
    angle_rates = 1.0 / (10000.0 ** depths)
    angle_rads = positions * angle_rates
    return np.concatenate([np.sin(angle_rads), np.cos(angle_rads)], axis=-1)


_PE = jnp.asarray(_positional_encoding(2048, D)[:SEQ], dtype=jnp.float32)


@functools.cache
def _make_kernel(n_batch):
    info = plsc.get_sparse_core_info()
    nc, ns = info.num_cores, info.num_subcores
    nw = nc * ns
    spw = n_batch // nw
    mesh = plsc.VectorSubcoreMesh(core_axis_name="c", subcore_axis_name="s")
    n_groups = (spw - 2) // NBUF

    @functools.partial(
        pl.kernel,
        out_type=jax.ShapeDtypeStruct((n_batch, SEQ, D), jnp.float32),
        mesh=mesh,
        scratch_types=[
            pltpu.VMEM((spw * SEQ,), jnp.int32),
            pltpu.VMEM((SEQ, D), jnp.float32),
        ] + [pltpu.VMEM((SEQ, D), jnp.float32)] * NBUF
          + [pltpu.SemaphoreType.DMA] * (2 * NBUF),
    )
    def k(x_hbm, table_hbm, pe_hbm, out_hbm, idx_v, pe_v,
          r0, r1, g0, g1, o0, o1):
        rows = (r0, r1)
        gsem = (g0, g1)
        osem = (o0, o1)
        wid = lax.axis_index("s") * nc + lax.axis_index("c")
        pltpu.sync_copy(x_hbm.at[pl.ds(wid * spw * SEQ, spw * SEQ)], idx_v)
        pltpu.sync_copy(pe_hbm, pe_v)
        row_base = wid * spw * SEQ

        def gather(u, b):
            return pltpu.make_async_copy(
                table_hbm.at[idx_v.at[pl.ds(u * SEQ, SEQ)]], rows[b],
                gsem[b])

        def out_cp(u, b):
            return pltpu.make_async_copy(
                rows[b], out_hbm.at[pl.ds(row_base + u * SEQ, SEQ)],
                osem[b])

        def compute(b):
            buf = rows[b]
            ng = D // 16

            def row_body(t, c):
                embs = [buf[t, pl.ds(g * 16, 16)] for g in range(ng)]
                pes = [pe_v[t, pl.ds(g * 16, 16)] for g in range(ng)]
                for g in range(ng):
                    buf[t, pl.ds(g * 16, 16)] = embs[g] * SCALE + pes[g]
                return c

            lax.fori_loop(0, SEQ, row_body, 0)

        def seq_body(u, carry):
            gather(u, 0).start()
            gather(u, 0).wait()
            compute(0)
            pltpu.sync_copy(rows[0], out_hbm.at[wid * spw + u])
            return carry

        lax.fori_loop(0, spw, seq_body, 0)

    return k


def kernel(x, table):
    n_batch, seq = x.shape
    out = _make_kernel(n_batch)(x.reshape(-1), table, _PE)
    return out.reshape(n_batch, seq, D)


# speedup vs baseline: 1.8223x; 1.0956x over previous
import functools
import math

import jax
import jax.numpy as jnp
import numpy as np
from jax import lax
from jax.experimental import pallas as pl
from jax.experimental.pallas import tpu as pltpu
from jax.experimental.pallas import tpu_sc as plsc

D = 128
SEQ = 200
NBUF = 2
SCALE = math.sqrt(float(D))


def _positional_encoding(length, depth):
    half = depth // 2
    positions = np.arange(length)[:, None].astype(np.float32)
    depths = np.arange(half)[None, :].astype(np.float32) / float(half)
    angle_rates = 1.0 / (10000.0 ** depths)
    angle_rads = positions * angle_rates
    return np.concatenate([np.sin(angle_rads), np.cos(angle_rads)], axis=-1)


_PE = jnp.asarray(_positional_encoding(2048, D)[:SEQ], dtype=jnp.float32)


@functools.cache
def _make_kernel(n_batch):
    info = plsc.get_sparse_core_info()
    nc, ns = info.num_cores, info.num_subcores
    nw = nc * ns
    spw = n_batch // nw
    mesh = plsc.VectorSubcoreMesh(core_axis_name="c", subcore_axis_name="s")
    n_groups = (spw - 2) // NBUF

    @functools.partial(
        pl.kernel,
        out_type=jax.ShapeDtypeStruct((n_batch, SEQ, D), jnp.float32),
        mesh=mesh,
        scratch_types=[
            pltpu.VMEM((spw * SEQ,), jnp.int32),
            pltpu.VMEM((SEQ, D), jnp.float32),
        ] + [pltpu.VMEM((SEQ, D), jnp.float32)] * NBUF
          + [pltpu.SemaphoreType.DMA] * (2 * NBUF),
    )
    def k(x_hbm, table_hbm, pe_hbm, out_hbm, idx_v, pe_v,
          r0, r1, g0, g1, o0, o1):
        rows = (r0, r1)
        gsem = (g0, g1)
        osem = (o0, o1)
        wid = lax.axis_index("s") * nc + lax.axis_index("c")
        pltpu.sync_copy(x_hbm.at[pl.ds(wid * spw * SEQ, spw * SEQ)], idx_v)
        pltpu.sync_copy(pe_hbm, pe_v)
        row_base = wid * spw * SEQ

        def gather(u, b):
            return pltpu.make_async_copy(
                table_hbm.at[idx_v.at[pl.ds(u * SEQ, SEQ)]], rows[b],
                gsem[b])

        def out_cp(u, b):
            return pltpu.make_async_copy(
                rows[b], out_hbm.at[wid * spw + u], osem[b])

        def compute(b):
            buf = rows[b]
            ng = D // 16

            def row_body(t, c):
                embs = [buf[t, pl.ds(g * 16, 16)] for g in range(ng)]
                pes = [pe_v[t, pl.ds(g * 16, 16)] for g in range(ng)]
                for g in range(ng):
                    buf[t, pl.ds(g * 16, 16)] = embs[g] * SCALE + pes[g]
                return c

            lax.fori_loop(0, SEQ, row_body, 0)

        def pair_body(p, carry):
            u = 2 * p
            gather(u, 0).start()
            gather(u + 1, 1).start()
            gather(u, 0).wait()
            compute(0)
            gather(u + 1, 1).wait()
            compute(1)
            out_cp(u, 0).start()
            out_cp(u + 1, 1).start()
            out_cp(u, 0).wait()
            out_cp(u + 1, 1).wait()
            return carry

        lax.fori_loop(0, spw // 2, pair_body, 0)

    return k


def kernel(x, table):
    n_batch, seq = x.shape
    out = _make_kernel(n_batch)(x.reshape(-1), table, _PE)
    return out.reshape(n_batch, seq, D)


# speedup vs baseline: 1.8523x; 1.0164x over previous
import functools
import math

import jax
import jax.numpy as jnp
import numpy as np
from jax import lax
from jax.experimental import pallas as pl
from jax.experimental.pallas import tpu as pltpu
from jax.experimental.pallas import tpu_sc as plsc

D = 128
SEQ = 200
NBUF = 3
SCALE = math.sqrt(float(D))


def _positional_encoding(length, depth):
    half = depth // 2
    positions = np.arange(length)[:, None].astype(np.float32)
    depths = np.arange(half)[None, :].astype(np.float32) / float(half)
    angle_rates = 1.0 / (10000.0 ** depths)
    angle_rads = positions * angle_rates
    return np.concatenate([np.sin(angle_rads), np.cos(angle_rads)], axis=-1)


_PE = jnp.asarray(_positional_encoding(2048, D)[:SEQ], dtype=jnp.float32)


@functools.cache
def _make_kernel(n_batch):
    info = plsc.get_sparse_core_info()
    nc, ns = info.num_cores, info.num_subcores
    nw = nc * ns
    spw = n_batch // nw
    mesh = plsc.VectorSubcoreMesh(core_axis_name="c", subcore_axis_name="s")
    n_groups = (spw - 2) // NBUF

    @functools.partial(
        pl.kernel,
        out_type=jax.ShapeDtypeStruct((n_batch, SEQ, D), jnp.float32),
        mesh=mesh,
        scratch_types=[
            pltpu.VMEM((spw * SEQ,), jnp.int32),
            pltpu.VMEM((SEQ, D), jnp.float32),
        ] + [pltpu.VMEM((SEQ, D), jnp.float32)] * NBUF
          + [pltpu.SemaphoreType.DMA] * (2 * NBUF),
    )
    def k(x_hbm, table_hbm, pe_hbm, out_hbm, idx_v, pe_v,
          r0, r1, r2, g0, g1, g2, o0, o1, o2):
        rows = (r0, r1, r2)
        gsem = (g0, g1, g2)
        osem = (o0, o1, o2)
        wid = lax.axis_index("s") * nc + lax.axis_index("c")
        pltpu.sync_copy(x_hbm.at[pl.ds(wid * spw * SEQ, spw * SEQ)], idx_v)
        pltpu.sync_copy(pe_hbm, pe_v)
        row_base = wid * spw * SEQ

        def gather(u, b):
            return pltpu.make_async_copy(
                table_hbm.at[idx_v.at[pl.ds(u * SEQ, SEQ)]], rows[b],
                gsem[b])

        def out_cp(u, b):
            return pltpu.make_async_copy(
                rows[b], out_hbm.at[wid * spw + u], osem[b])

        def compute(b):
            buf = rows[b]
            ng = D // 16

            def row_body(t, c):
                embs = [buf[t, pl.ds(g * 16, 16)] for g in range(ng)]
                pes = [pe_v[t, pl.ds(g * 16, 16)] for g in range(ng)]
                for g in range(ng):
                    buf[t, pl.ds(g * 16, 16)] = embs[g] * SCALE + pes[g]
                return c

            lax.fori_loop(0, SEQ, row_body, 0)

        def triple_body(p, carry):
            u = NBUF * p
            for j in range(NBUF):
                gather(u + j, j).start()
            for j in range(NBUF):
                gather(u + j, j).wait()
                compute(j)
            for j in range(NBUF):
                out_cp(u + j, j).start()
            for j in range(NBUF):
                out_cp(u + j, j).wait()
            return carry

        n_full = spw // NBUF
        lax.fori_loop(0, n_full, triple_body, 0)
        for j in range(spw - n_full * NBUF):
            u = n_full * NBUF + j
            gather(u, j).start()
        for j in range(spw - n_full * NBUF):
            u = n_full * NBUF + j
            gather(u, j).wait()
            compute(j)
            out_cp(u, j).start()
        for j in range(spw - n_full * NBUF):
            u = n_full * NBUF + j
            out_cp(u, j).wait()

    return k


def kernel(x, table):
    n_batch, seq = x.shape
    out = _make_kernel(n_batch)(x.reshape(-1), table, _PE)
    return out.reshape(n_batch, seq, D)
